# baseline (speedup 1.0000x reference)
import functools

import jax
import jax.numpy as jnp
from jax import lax
from jax.experimental import pallas as pl
from jax.experimental.pallas import tpu as pltpu
from jax.experimental.pallas import tpu_sc as plsc

N_NODES = 10000
N_EDGES = 320000
D = 128

N_PAD = 10240
E_PAD = 327680
K = 128
CHUNKS_PER_TILE = 80
CHUNKS_PER_HALF = 16
TOTAL_HALVES = 10
CORE0_HALVES = 10
ROWS_PER_TILE = N_PAD // 16


def _sc_aggregate(xp, rowc, colc, zeros_f, zeros_d, ones_k):
    mesh = plsc.VectorSubcoreMesh(
        core_axis_name="c", subcore_axis_name="s", num_cores=2)

    @functools.partial(
        pl.kernel,
        mesh=mesh,
        out_type=[
            jax.ShapeDtypeStruct((2, N_PAD, D), jnp.float32),
            jax.ShapeDtypeStruct((2, N_PAD), jnp.float32),
        ],
        scratch_types=[
            pltpu.VMEM_SHARED((N_PAD, D), jnp.float32),
            pltpu.VMEM_SHARED((N_PAD,), jnp.float32),
            pltpu.VMEM((CHUNKS_PER_HALF, K), jnp.int32),
            pltpu.VMEM((CHUNKS_PER_HALF, K), jnp.int32),
            pltpu.VMEM((K, D), jnp.float32),
            pltpu.VMEM((K, D), jnp.float32),
            pltpu.VMEM((K,), jnp.float32),
            pltpu.SemaphoreType.DMA,
            pltpu.SemaphoreType.DMA,
            pltpu.SemaphoreType.DMA,
            pltpu.SemaphoreType.DMA,
            pltpu.SemaphoreType.DMA,
        ],
    )
    def sc_kernel(x_hbm, row_hbm, col_hbm, zf_hbm, zd_hbm, ones_hbm,
                  acc_out, deg_out,
                  acc_sh, deg_sh, ridx_v, cidx_v,
                  rows0, rows1, ones_v,
                  sg0, sg1, ss0, ss1, sd):
        c = lax.axis_index("c")
        s = lax.axis_index("s")
        wid = c * 16 + s
        bufs = (rows0, rows1)
        sgs = (sg0, sg1)
        sss = (ss0, ss1)

        pltpu.sync_copy(zf_hbm, acc_sh.at[pl.ds(s * ROWS_PER_TILE, ROWS_PER_TILE)])
        pltpu.sync_copy(zd_hbm, deg_sh.at[pl.ds(s * ROWS_PER_TILE, ROWS_PER_TILE)])
        pltpu.sync_copy(ones_hbm, ones_v)
        plsc.subcore_barrier()

        def start_gather(j, b):
            pltpu.async_copy(x_hbm.at[ridx_v.at[j]], bufs[b], sgs[b])

        def wait_gather(b):
            pltpu.make_async_copy(x_hbm.at[ridx_v.at[0]], bufs[b], sgs[b]).wait()

        def start_scatter(j, b):
            pltpu.async_copy(bufs[b], acc_sh.at[cidx_v.at[j]], sss[b], add=True)

        def wait_scatter(b):
            pltpu.make_async_copy(
                bufs[b], acc_sh.at[cidx_v.at[0]], sss[b]).wait()

        def start_deg(j):
            pltpu.async_copy(ones_v, deg_sh.at[ridx_v.at[j]], sd, add=True)

        def wait_deg():
            pltpu.make_async_copy(ones_v, deg_sh.at[ridx_v.at[0]], sd).wait()

        nh = jnp.where(c == 0, CORE0_HALVES, TOTAL_HALVES - CORE0_HALVES)
        tile_start = jnp.where(
            c == 0, s * CORE0_HALVES,
            16 * CORE0_HALVES + s * (TOTAL_HALVES - CORE0_HALVES))

        def half_body(h, carry):
            base = (tile_start + h) * CHUNKS_PER_HALF
            pltpu.sync_copy(row_hbm.at[pl.ds(base, CHUNKS_PER_HALF)], ridx_v)
            pltpu.sync_copy(col_hbm.at[pl.ds(base, CHUNKS_PER_HALF)], cidx_v)
            start_gather(0, 0)

            def body(t, carry2):
                j = t * 2
                def _wait_prev():
                    wait_scatter(1)
                    wait_deg()
                pl.when(t >= 1)(_wait_prev)
                start_gather(j + 1, 1)
                wait_gather(0)
                start_scatter(j, 0)
                start_deg(j)
                wait_scatter(0)
                wait_deg()
                pl.when(t < CHUNKS_PER_HALF // 2 - 1)(
                    lambda: start_gather(j + 2, 0))
                wait_gather(1)
                start_scatter(j + 1, 1)
                start_deg(j + 1)
                return carry2

            lax.fori_loop(0, CHUNKS_PER_HALF // 2, body, 0)
            wait_scatter(1)
            wait_deg()
            return carry

        lax.fori_loop(0, nh, half_body, 0)
        plsc.subcore_barrier()

        sl = pl.ds(s * ROWS_PER_TILE, ROWS_PER_TILE)
        pltpu.sync_copy(acc_sh.at[sl], acc_out.at[c, sl])
        pltpu.sync_copy(deg_sh.at[sl], deg_out.at[c, sl])

    return sc_kernel(xp, rowc, colc, zeros_f, zeros_d, ones_k)


def _tc_finish(acc2, deg2t, xp, W, b2):
    BR = 256
    grid = N_PAD // BR

    def body(p_ref, d_ref, x_ref, w_ref, b_ref, o_ref):
        acc = p_ref[0] + p_ref[1]
        d = d_ref[...]
        deg = jnp.maximum(d[:, 0:1] + d[:, 1:2], 1.0)
        h = acc / deg + x_ref[...]
        o_ref[...] = lax.dot_general(
            h, w_ref[...], (((1,), (1,)), ((), ())),
            preferred_element_type=jnp.float32) + b_ref[...]

    return pl.pallas_call(
        body,
        grid=(grid,),
        in_specs=[
            pl.BlockSpec((2, BR, D), lambda i: (0, i, 0)),
            pl.BlockSpec((BR, 2), lambda i: (i, 0)),
            pl.BlockSpec((BR, D), lambda i: (i, 0)),
            pl.BlockSpec((D, D), lambda i: (0, 0)),
            pl.BlockSpec((1, D), lambda i: (0, 0)),
        ],
        out_specs=pl.BlockSpec((BR, D), lambda i: (i, 0)),
        out_shape=jax.ShapeDtypeStruct((N_PAD, D), jnp.float32),
    )(acc2, deg2t, xp, W, b2)


def kernel(x, edge_index, W, b):
    xp = jnp.zeros((N_PAD, D), jnp.float32).at[:N_NODES].set(x)
    row = edge_index[0]
    col = edge_index[1]
    rowp = jnp.full((E_PAD,), N_NODES, jnp.int32).at[:N_EDGES].set(row)
    colp = jnp.full((E_PAD,), N_NODES, jnp.int32).at[:N_EDGES].set(col)
    rowc = rowp.reshape(E_PAD // K, K)
    colc = colp.reshape(E_PAD // K, K)

    zeros_f = jnp.zeros((ROWS_PER_TILE, D), jnp.float32)
    zeros_d = jnp.zeros((ROWS_PER_TILE,), jnp.float32)
    ones_k = jnp.ones((K,), jnp.float32)

    acc2, deg2 = _sc_aggregate(xp, rowc, colc, zeros_f, zeros_d, ones_k)
    out = _tc_finish(acc2, deg2.T, xp, W, b.reshape(1, D))
    return out[:N_NODES]


# speedup vs baseline: 1.1963x; 1.1963x over previous
import functools

import jax
import jax.numpy as jnp
from jax import lax
from jax.experimental import pallas as pl
from jax.experimental.pallas import tpu as pltpu
from jax.experimental.pallas import tpu_sc as plsc

N_NODES = 10000
N_EDGES = 320000
D = 128

N_PAD = 10240
E_PAD = 327680
K = 128
CHUNKS_PER_TILE = 80
CHUNKS_PER_HALF = 16
TOTAL_HALVES = 10
CORE0_HALVES = 9
ROWS_PER_TILE = N_PAD // 16


def _sc_aggregate(xp, rowc, colc, zeros_f, zeros_d, ones_k):
    mesh = plsc.VectorSubcoreMesh(
        core_axis_name="c", subcore_axis_name="s", num_cores=2)

    @functools.partial(
        pl.kernel,
        mesh=mesh,
        out_type=[
            jax.ShapeDtypeStruct((2, N_PAD, D), jnp.float32),
            jax.ShapeDtypeStruct((2, N_PAD), jnp.float32),
        ],
        scratch_types=[
            pltpu.VMEM_SHARED((N_PAD, D), jnp.float32),
            pltpu.VMEM_SHARED((N_PAD,), jnp.float32),
            pltpu.VMEM((CHUNKS_PER_HALF, K), jnp.int32),
            pltpu.VMEM((CHUNKS_PER_HALF, K), jnp.int32),
            pltpu.VMEM((K, D), jnp.float32),
            pltpu.VMEM((K, D), jnp.float32),
            pltpu.VMEM((K,), jnp.float32),
            pltpu.SemaphoreType.DMA,
            pltpu.SemaphoreType.DMA,
            pltpu.SemaphoreType.DMA,
            pltpu.SemaphoreType.DMA,
            pltpu.SemaphoreType.DMA,
        ],
    )
    def sc_kernel(x_hbm, row_hbm, col_hbm, zf_hbm, zd_hbm, ones_hbm,
                  acc_out, deg_out,
                  acc_sh, deg_sh, ridx_v, cidx_v,
                  rows0, rows1, ones_v,
                  sg0, sg1, ss0, ss1, sd):
        c = lax.axis_index("c")
        s = lax.axis_index("s")
        wid = c * 16 + s
        bufs = (rows0, rows1)
        sgs = (sg0, sg1)
        sss = (ss0, ss1)

        pltpu.sync_copy(zf_hbm, acc_sh.at[pl.ds(s * ROWS_PER_TILE, ROWS_PER_TILE)])
        pltpu.sync_copy(zd_hbm, deg_sh.at[pl.ds(s * ROWS_PER_TILE, ROWS_PER_TILE)])
        pltpu.sync_copy(ones_hbm, ones_v)
        plsc.subcore_barrier()

        def start_gather(j, b):
            pltpu.async_copy(x_hbm.at[ridx_v.at[j]], bufs[b], sgs[b])

        def wait_gather(b):
            pltpu.make_async_copy(x_hbm.at[ridx_v.at[0]], bufs[b], sgs[b]).wait()

        def start_scatter(j, b):
            pltpu.async_copy(bufs[b], acc_sh.at[cidx_v.at[j]], sss[b], add=True)

        def wait_scatter(b):
            pltpu.make_async_copy(
                bufs[b], acc_sh.at[cidx_v.at[0]], sss[b]).wait()

        def start_deg(j):
            pltpu.async_copy(ones_v, deg_sh.at[ridx_v.at[j]], sd, add=True)

        def wait_deg():
            pltpu.make_async_copy(ones_v, deg_sh.at[ridx_v.at[0]], sd).wait()

        nh = jnp.where(c == 0, CORE0_HALVES, TOTAL_HALVES - CORE0_HALVES)
        tile_start = jnp.where(
            c == 0, s * CORE0_HALVES,
            16 * CORE0_HALVES + s * (TOTAL_HALVES - CORE0_HALVES))

        def half_body(h, carry):
            base = (tile_start + h) * CHUNKS_PER_HALF
            pltpu.sync_copy(row_hbm.at[pl.ds(base, CHUNKS_PER_HALF)], ridx_v)
            pltpu.sync_copy(col_hbm.at[pl.ds(base, CHUNKS_PER_HALF)], cidx_v)
            start_gather(0, 0)

            def body(t, carry2):
                j = t * 2
                def _wait_prev():
                    wait_scatter(1)
                    wait_deg()
                pl.when(t >= 1)(_wait_prev)
                start_gather(j + 1, 1)
                wait_gather(0)
                start_scatter(j, 0)
                start_deg(j)
                wait_scatter(0)
                wait_deg()
                pl.when(t < CHUNKS_PER_HALF // 2 - 1)(
                    lambda: start_gather(j + 2, 0))
                wait_gather(1)
                start_scatter(j + 1, 1)
                start_deg(j + 1)
                return carry2

            lax.fori_loop(0, CHUNKS_PER_HALF // 2, body, 0)
            wait_scatter(1)
            wait_deg()
            return carry

        lax.fori_loop(0, nh, half_body, 0)
        plsc.subcore_barrier()

        sl = pl.ds(s * ROWS_PER_TILE, ROWS_PER_TILE)
        pltpu.sync_copy(acc_sh.at[sl], acc_out.at[c, sl])
        pltpu.sync_copy(deg_sh.at[sl], deg_out.at[c, sl])

    return sc_kernel(xp, rowc, colc, zeros_f, zeros_d, ones_k)


def _tc_finish(acc2, deg2t, xp, W, b2):
    BR = 256
    grid = N_PAD // BR

    def body(p_ref, d_ref, x_ref, w_ref, b_ref, o_ref):
        acc = p_ref[0] + p_ref[1]
        d = d_ref[...]
        deg = jnp.maximum(d[:, 0:1] + d[:, 1:2], 1.0)
        h = acc / deg + x_ref[...]
        o_ref[...] = lax.dot_general(
            h, w_ref[...], (((1,), (1,)), ((), ())),
            preferred_element_type=jnp.float32) + b_ref[...]

    return pl.pallas_call(
        body,
        grid=(grid,),
        in_specs=[
            pl.BlockSpec((2, BR, D), lambda i: (0, i, 0)),
            pl.BlockSpec((BR, 2), lambda i: (i, 0)),
            pl.BlockSpec((BR, D), lambda i: (i, 0)),
            pl.BlockSpec((D, D), lambda i: (0, 0)),
            pl.BlockSpec((1, D), lambda i: (0, 0)),
        ],
        out_specs=pl.BlockSpec((BR, D), lambda i: (i, 0)),
        out_shape=jax.ShapeDtypeStruct((N_PAD, D), jnp.float32),
    )(acc2, deg2t, xp, W, b2)


def kernel(x, edge_index, W, b):
    xp = jnp.zeros((N_PAD, D), jnp.float32).at[:N_NODES].set(x)
    row = edge_index[0]
    col = edge_index[1]
    rowp = jnp.full((E_PAD,), N_NODES, jnp.int32).at[:N_EDGES].set(row)
    colp = jnp.full((E_PAD,), N_NODES, jnp.int32).at[:N_EDGES].set(col)
    rowc = rowp.reshape(E_PAD // K, K)
    colc = colp.reshape(E_PAD // K, K)

    zeros_f = jnp.zeros((ROWS_PER_TILE, D), jnp.float32)
    zeros_d = jnp.zeros((ROWS_PER_TILE,), jnp.float32)
    ones_k = jnp.ones((K,), jnp.float32)

    acc2, deg2 = _sc_aggregate(xp, rowc, colc, zeros_f, zeros_d, ones_k)
    out = _tc_finish(acc2, deg2.T, xp, W, b.reshape(1, D))
    return out[:N_NODES]
